# Initial kernel scaffold; baseline (speedup 1.0000x reference)
import jax
import jax.numpy as jnp
from jax.experimental import pallas as pl


def kernel(drug_feat, disease_feat, edge_dd, edge_rd, edge_rr, mp_ins, llm_rep, W_lin_drug, b_lin_drug, W_lin_dis, b_lin_dis, W_dd, W_rd, W_rr, W_self_drug, W_self_dis, Wa_drug, va_drug, Wa_dis, va_dis, W_agg, V_mil, w_mil, W_llm, b_llm, W_mlp1, b_mlp1, W_mlp2, b_mlp2):
    raise NotImplementedError("write your pallas kernel here")


# speedup vs baseline: 2.2053x; 2.2053x over previous
import functools

import jax
import jax.numpy as jnp
from jax import lax
from jax.experimental import pallas as pl
from jax.experimental.pallas import tpu as pltpu
from jax.experimental.pallas import tpu_sc as plsc

_N = 25000
_NP = 25088
_RPT = _NP // 16
_D = 128
_HD = 64
_E = 400000
_CH = 128
_EP = 401408
_EPT = _EP // 16
_NCH = _EPT // _CH
_NB = 1024
_BAG = 16
_LLM_D = 32000
_KB = 3200
_NKB = _LLM_D // _KB

def _get_mesh():
    return plsc.VectorSubcoreMesh(core_axis_name="c", subcore_axis_name="s",
                                  num_cores=2, num_subcores=16)


@functools.lru_cache(maxsize=None)
def _make_segsum(n_phases):
    n_in = 2 + 4 * n_phases

    @functools.partial(
        pl.kernel,
        out_type=[jax.ShapeDtypeStruct((_NP, _HD), jnp.float32),
                  jax.ShapeDtypeStruct((_NP, _HD), jnp.float32)],
        mesh=_get_mesh(),
        compiler_params=pltpu.CompilerParams(use_tc_tiling_on_sc=False),
        scratch_types=[
            pltpu.VMEM((_CH,), jnp.int32),
            pltpu.VMEM((_CH,), jnp.int32),
            pltpu.VMEM((_CH, _HD), jnp.float32),
            pltpu.VMEM_SHARED((_NP, _HD), jnp.float32),
            pltpu.SemaphoreType.DMA,
        ],
    )
    def segsum(*refs):
        ins = refs[:n_in]
        out_l, out_r = refs[n_in], refs[n_in + 1]
        src_v, dst_v, rows_v, acc, sem = refs[n_in + 2:]
        init_l, init_r = ins[0], ins[1]
        c = lax.axis_index("c")
        t = lax.axis_index("s")
        r0 = t * _RPT

        def run(init, tabs, out):
            pltpu.sync_copy(init.at[pl.ds(r0, _RPT)], acc.at[pl.ds(r0, _RPT)])
            plsc.subcore_barrier()
            for ph in range(n_phases):
                tab = tabs[ph]
                src = ins[2 + 4 * ph + 2]
                dst = ins[2 + 4 * ph + 3]

                def chunk(i, _):
                    off = t * _EPT + i * _CH
                    pltpu.sync_copy(src.at[pl.ds(off, _CH)], src_v)
                    pltpu.sync_copy(dst.at[pl.ds(off, _CH)], dst_v)
                    pltpu.async_copy(tab.at[src_v], rows_v, sem).wait()
                    pltpu.sync_copy(rows_v, acc.at[dst_v], add=True)
                    return 0

                lax.fori_loop(0, _NCH, chunk, 0)
            plsc.subcore_barrier()
            pltpu.sync_copy(acc.at[pl.ds(r0, _RPT)], out.at[pl.ds(r0, _RPT)])

        @pl.when(c == 0)
        def _():
            run(init_l, [ins[2 + 4 * p] for p in range(n_phases)], out_l)

        @pl.when(c == 1)
        def _():
            run(init_r, [ins[2 + 4 * p + 1] for p in range(n_phases)], out_r)

    return segsum


_MPW = (_NB * _BAG) // 32
_MPCH = _MPW // _CH


@functools.lru_cache(maxsize=None)
def _make_mp_gather():
    @functools.partial(
        pl.kernel,
        out_type=[jax.ShapeDtypeStruct((_NB * _BAG, _D), jnp.float32),
                  jax.ShapeDtypeStruct((_NB * _BAG, _D), jnp.float32)],
        mesh=_get_mesh(),
        scratch_types=[
            pltpu.VMEM((_CH,), jnp.int32),
            pltpu.VMEM((_CH, _D), jnp.float32),
            pltpu.SemaphoreType.DMA,
        ],
    )
    def mp_gather(hdf, hsf, idxd, idxs, gd, gs, idx_v, rows_v, sem):
        c = lax.axis_index("c")
        s = lax.axis_index("s")
        wid = s * 2 + c

        def one(tab, idx, out):
            def chunk(i, _):
                off = wid * _MPW + i * _CH
                pltpu.sync_copy(idx.at[pl.ds(off, _CH)], idx_v)
                pltpu.async_copy(tab.at[idx_v], rows_v, sem).wait()
                pltpu.sync_copy(rows_v, out.at[pl.ds(off, _CH)])
                return 0

            lax.fori_loop(0, _MPCH, chunk, 0)

        one(hdf, idxd, gd)
        one(hsf, idxs, gs)

    return mp_gather


def _dot(a, b):
    return jnp.dot(a, b, preferred_element_type=jnp.float32)


def _lin_body(xd, xs, wd, bd, ws, bs, od, os_):
    od[...] = jnp.maximum(_dot(xd[...], wd[...]) + bd[...], 0.0)
    os_[...] = jnp.maximum(_dot(xs[...], ws[...]) + bs[...], 0.0)


def _row_spec(r, ncols=_D):
    return pl.BlockSpec((r, ncols), lambda i: (i, 0))


def _full_spec(shape):
    return pl.BlockSpec(shape, lambda i: tuple(0 for _ in shape))


def _lin(xd, xs, wd, bd, ws, bs):
    return pl.pallas_call(
        _lin_body,
        grid=(16,),
        in_specs=[_row_spec(_RPT), _row_spec(_RPT),
                  _full_spec((_D, _D)), _full_spec((1, _D)),
                  _full_spec((_D, _D)), _full_spec((1, _D))],
        out_specs=[_row_spec(_RPT), _row_spec(_RPT)],
        out_shape=[jax.ShapeDtypeStruct((_NP, _D), jnp.float32)] * 2,
    )(xd, xs, wd, bd, ws, bs)


def _mm_body(hd, hs, wdd, wrd, wrr, wsd, wss, *outs):
    d = hd[...]
    s = hs[...]
    vals = [_dot(s, wdd[...]), _dot(d, wrd[...]), _dot(d, wrr[...]),
            _dot(d, wsd[...]), _dot(s, wss[...])]
    for j, v in enumerate(vals):
        outs[2 * j][...] = v[:, :_HD]
        outs[2 * j + 1][...] = v[:, _HD:]


def _layer_mm(hd, hs, wdd, wrd, wrr, wsd, wss):
    return pl.pallas_call(
        _mm_body,
        grid=(16,),
        in_specs=[_row_spec(_RPT), _row_spec(_RPT)] + [_full_spec((_D, _D))] * 5,
        out_specs=[_row_spec(_RPT, _HD)] * 10,
        out_shape=[jax.ShapeDtypeStruct((_NP, _HD), jnp.float32)] * 10,
    )(hd, hs, wdd, wrd, wrr, wsd, wss)


def _relu_body(al, ar, bl, br, oa, ob):
    oa[...] = jnp.concatenate(
        [jnp.maximum(al[...], 0.0), jnp.maximum(ar[...], 0.0)], axis=1)
    ob[...] = jnp.concatenate(
        [jnp.maximum(bl[...], 0.0), jnp.maximum(br[...], 0.0)], axis=1)


def _relu_cat(al, ar, bl, br):
    return pl.pallas_call(
        _relu_body,
        grid=(16,),
        in_specs=[_row_spec(_RPT, _HD)] * 4,
        out_specs=[_row_spec(_RPT)] * 2,
        out_shape=[jax.ShapeDtypeStruct((_NP, _D), jnp.float32)] * 2,
    )(al, ar, bl, br)


def _attn_pool(h0, h1, h2, wa, va):
    hs = [h0[...], h1[...], h2[...]]
    va_col = va[...].reshape(_D, 1)
    es = [_dot(jnp.tanh(_dot(h, wa[...])), va_col) for h in hs]
    m = jnp.maximum(jnp.maximum(es[0], es[1]), es[2])
    ws = [jnp.exp(e - m) for e in es]
    tot = ws[0] + ws[1] + ws[2]
    return (ws[0] * hs[0] + ws[1] * hs[1] + ws[2] * hs[2]) / tot


def _attn_body(d0, d1, d2, s0, s1, s2, wad, vad, was, vas, od, os_):
    od[...] = _attn_pool(d0, d1, d2, wad, vad)
    os_[...] = _attn_pool(s0, s1, s2, was, vas)


def _layer_attn(d0, d1, d2, s0, s1, s2, wad, vad, was, vas):
    return pl.pallas_call(
        _attn_body,
        grid=(16,),
        in_specs=[_row_spec(_RPT)] * 6
        + [_full_spec((_D, _D)), _full_spec((1, _D)),
           _full_spec((_D, _D)), _full_spec((1, _D))],
        out_specs=[_row_spec(_RPT)] * 2,
        out_shape=[jax.ShapeDtypeStruct((_NP, _D), jnp.float32)] * 2,
    )(d0, d1, d2, s0, s1, s2, wad, vad, was, vas)


_MB = 256


def _mil_body(gd, gs, wagg, vmil, wmil, attn_o, bag_o):
    g = (gd[...] + gs[...]).reshape(_MB * _BAG, _D)
    ins = jnp.maximum(_dot(g, wagg[...]), 0.0)
    t3 = jnp.tanh(_dot(ins, vmil[...])).reshape(_MB, _BAG, _D)
    ins3 = ins.reshape(_MB, _BAG, _D)
    w_col = wmil[...].reshape(_D, 1)
    cols = [_dot(t3[:, k, :], w_col) for k in range(_BAG)]
    al = jnp.concatenate(cols, axis=1)
    m = jnp.max(al, axis=1, keepdims=True)
    e = jnp.exp(al - m)
    attn = e / jnp.sum(e, axis=1, keepdims=True)
    attn_o[...] = attn
    bag = attn[:, 0:1] * ins3[:, 0, :]
    for k in range(1, _BAG):
        bag = bag + attn[:, k:k + 1] * ins3[:, k, :]
    bag_o[...] = bag


def _mil(gd3, gs3, wagg, vmil, wmil):
    return pl.pallas_call(
        _mil_body,
        grid=(_NB // _MB,),
        in_specs=[pl.BlockSpec((_MB, _BAG, _D), lambda i: (i, 0, 0))] * 2
        + [_full_spec((_D, _D)), _full_spec((_D, _D)), _full_spec((1, _D))],
        out_specs=[pl.BlockSpec((_MB, _BAG), lambda i: (i, 0)),
                   pl.BlockSpec((_MB, _D), lambda i: (i, 0))],
        out_shape=[jax.ShapeDtypeStruct((_NB, _BAG), jnp.float32),
                   jax.ShapeDtypeStruct((_NB, _D), jnp.float32)],
    )(gd3, gs3, wagg, vmil, wmil)


def _llm_body(x, w, b, o):
    k = pl.program_id(0)

    @pl.when(k == 0)
    def _():
        o[...] = jnp.broadcast_to(b[...], (_NB, _D))

    o[...] += _dot(x[...], w[...])

    @pl.when(k == _NKB - 1)
    def _():
        y = o[...]
        n = jnp.sqrt(jnp.sum(y * y, axis=1, keepdims=True)) + 1e-12
        o[...] = y / n


def _llm_head(x, w, b):
    return pl.pallas_call(
        _llm_body,
        grid=(_NKB,),
        in_specs=[pl.BlockSpec((_NB, _KB), lambda k: (0, k)),
                  pl.BlockSpec((_KB, _D), lambda k: (k, 0)),
                  _full_spec((1, _D))],
        out_specs=pl.BlockSpec((_NB, _D), lambda k: (0, 0)),
        out_shape=jax.ShapeDtypeStruct((_NB, _D), jnp.float32),
    )(x, w, b)


def _head_body(bag, llm_n, w1, b1, w2, b2, o):
    kg = bag[...]
    n = jnp.sqrt(jnp.sum(kg * kg, axis=1, keepdims=True)) + 1e-12
    kgn = kg / n
    w1v = w1[...]
    h = _dot(kgn, w1v[:_D, :]) + _dot(llm_n[...], w1v[_D:, :]) + b1[...]
    h = jnp.maximum(h, 0.0)
    pred = _dot(h, w2[...]) + b2[0, 0]
    o[...] = jnp.broadcast_to(pred, (_NB, _D))


def _head(bag, llm_n, w1, b1, w2, b2):
    return pl.pallas_call(
        _head_body,
        grid=(1,),
        in_specs=[_full_spec((_NB, _D)), _full_spec((_NB, _D)),
                  _full_spec((2 * _D, _D)), _full_spec((1, _D)),
                  _full_spec((_D, 1)), _full_spec((1, 1))],
        out_specs=_full_spec((_NB, _D)),
        out_shape=jax.ShapeDtypeStruct((_NB, _D), jnp.float32),
    )(bag, llm_n, w1, b1, w2, b2)


def kernel(drug_feat, disease_feat, edge_dd, edge_rd, edge_rr, mp_ins,
           llm_rep, W_lin_drug, b_lin_drug, W_lin_dis, b_lin_dis, W_dd,
           W_rd, W_rr, W_self_drug, W_self_dis, Wa_drug, va_drug, Wa_dis,
           va_dis, W_agg, V_mil, w_mil, W_llm, b_llm, W_mlp1, b_mlp1,
           W_mlp2, b_mlp2):
    rowpad = ((0, _NP - _N), (0, 0))
    dfp = jnp.pad(drug_feat, rowpad)
    sfp = jnp.pad(disease_feat, rowpad)

    def pad_edges(e):
        src = jnp.pad(e[0], (0, _EP - _E))
        dst = jnp.pad(e[1], (0, _EP - _E), constant_values=_N + 80)
        return src, dst

    dd_s, dd_d = pad_edges(edge_dd)
    rd_s, rd_d = pad_edges(edge_rd)
    rr_s, rr_d = pad_edges(edge_rr)

    row = lambda v: v.reshape(1, -1)

    hd, hs = _lin(dfp, sfp, W_lin_drug, row(b_lin_drug),
                  W_lin_dis, row(b_lin_dis))
    drugs = [hd]
    diss = [hs]
    for l in range(2):
        (tddL, tddR, trdL, trdR, trrL, trrR,
         sdL, sdR, ssL, ssR) = _layer_mm(
            hd, hs, W_dd[l], W_rd[l], W_rr[l], W_self_drug[l], W_self_dis[l])
        msL, msR = _make_segsum(2)(ssL, ssR, tddL, tddR, dd_s, dd_d,
                                   trdL, trdR, rd_s, rd_d)
        mdL, mdR = _make_segsum(1)(sdL, sdR, trrL, trrR, rr_s, rr_d)
        hd, hs = _relu_cat(mdL, mdR, msL, msR)
        drugs.append(hd)
        diss.append(hs)

    hdf, hsf = _layer_attn(drugs[0], drugs[1], drugs[2],
                           diss[0], diss[1], diss[2],
                           Wa_drug, row(va_drug), Wa_dis, row(va_dis))

    idxd = mp_ins[..., 0].reshape(-1)
    idxs = mp_ins[..., 1].reshape(-1)
    gd, gs = _make_mp_gather()(hdf, hsf, idxd, idxs)

    attn, bag = _mil(gd.reshape(_NB, _BAG, _D), gs.reshape(_NB, _BAG, _D),
                     W_agg, V_mil, row(w_mil))

    llm_n = _llm_head(llm_rep, W_llm, row(b_llm))
    pred_full = _head(bag, llm_n, W_mlp1, row(b_mlp1), W_mlp2,
                      b_mlp2.reshape(1, 1))
    return pred_full[:, :1], attn


# speedup vs baseline: 3.0249x; 1.3717x over previous
import functools

import jax
import jax.numpy as jnp
from jax import lax
from jax.experimental import pallas as pl
from jax.experimental.pallas import tpu as pltpu
from jax.experimental.pallas import tpu_sc as plsc

_N = 25000
_NP = 25088
_RPT = _NP // 16
_D = 128
_HD = 64
_E = 400000
_CH = 128
_EP = 401408
_EPT = _EP // 16
_NCH = _EPT // _CH
_NB = 1024
_BAG = 16
_LLM_D = 32000
_KB = 3200
_NKB = _LLM_D // _KB

def _get_mesh():
    return plsc.VectorSubcoreMesh(core_axis_name="c", subcore_axis_name="s",
                                  num_cores=2, num_subcores=16)


@functools.lru_cache(maxsize=None)
def _make_segsum(n_phases):
    n_in = 2 + 4 * n_phases
    nbuf = 2
    sec = 14
    nsec = _NCH // sec

    @functools.partial(
        pl.kernel,
        out_type=[jax.ShapeDtypeStruct((_NP, _HD), jnp.float32),
                  jax.ShapeDtypeStruct((_NP, _HD), jnp.float32)],
        mesh=_get_mesh(),
        compiler_params=pltpu.CompilerParams(use_tc_tiling_on_sc=False),
        scratch_types=[
            pltpu.VMEM((sec, _CH), jnp.int32),
            pltpu.VMEM((sec, _CH), jnp.int32),
        ]
        + [pltpu.VMEM((_CH, _HD), jnp.float32) for _ in range(nbuf)]
        + [pltpu.SemaphoreType.DMA for _ in range(nbuf)]
        + [pltpu.VMEM_SHARED((_NP, _HD), jnp.float32)],
    )
    def segsum(*refs):
        ins = refs[:n_in]
        out_l, out_r = refs[n_in], refs[n_in + 1]
        src_v, dst_v = refs[n_in + 2], refs[n_in + 3]
        bufs = refs[n_in + 4:n_in + 4 + nbuf]
        sems = refs[n_in + 4 + nbuf:n_in + 4 + 2 * nbuf]
        acc = refs[n_in + 4 + 2 * nbuf]
        init_l, init_r = ins[0], ins[1]
        c = lax.axis_index("c")
        t = lax.axis_index("s")
        r0 = t * _RPT

        def run(init, tabs, out):
            pltpu.sync_copy(init.at[pl.ds(r0, _RPT)], acc.at[pl.ds(r0, _RPT)])
            plsc.subcore_barrier()
            for ph in range(n_phases):
                tab = tabs[ph]
                src = ins[2 + 4 * ph + 2]
                dst = ins[2 + 4 * ph + 3]

                def section(s, _):
                    pltpu.sync_copy(src.at[t, pl.ds(s * sec, sec)], src_v)
                    pltpu.sync_copy(dst.at[t, pl.ds(s * sec, sec)], dst_v)

                    def chunk(k, _):
                        pltpu.async_copy(tab.at[src_v.at[k]], bufs[0],
                                         sems[0]).wait()
                        pltpu.sync_copy(bufs[0], acc.at[dst_v.at[k]],
                                        add=True)
                        return 0

                    lax.fori_loop(0, sec, chunk, 0)
                    return 0

                lax.fori_loop(0, nsec, section, 0)
            plsc.subcore_barrier()
            pltpu.sync_copy(acc.at[pl.ds(r0, _RPT)], out.at[pl.ds(r0, _RPT)])

        @pl.when(c == 0)
        def _():
            run(init_l, [ins[2 + 4 * p] for p in range(n_phases)], out_l)

        @pl.when(c == 1)
        def _():
            run(init_r, [ins[2 + 4 * p + 1] for p in range(n_phases)], out_r)

    return segsum


_MPW = (_NB * _BAG) // 32
_MPCH = _MPW // _CH


@functools.lru_cache(maxsize=None)
def _make_mp_gather():
    @functools.partial(
        pl.kernel,
        out_type=[jax.ShapeDtypeStruct((_NB * _BAG, _D), jnp.float32),
                  jax.ShapeDtypeStruct((_NB * _BAG, _D), jnp.float32)],
        mesh=_get_mesh(),
        scratch_types=[
            pltpu.VMEM((_CH,), jnp.int32),
            pltpu.VMEM((_CH, _D), jnp.float32),
            pltpu.SemaphoreType.DMA,
        ],
    )
    def mp_gather(hdf, hsf, idxd, idxs, gd, gs, idx_v, rows_v, sem):
        c = lax.axis_index("c")
        s = lax.axis_index("s")
        wid = s * 2 + c

        def one(tab, idx, out):
            def chunk(i, _):
                off = wid * _MPW + i * _CH
                pltpu.sync_copy(idx.at[pl.ds(off, _CH)], idx_v)
                pltpu.async_copy(tab.at[idx_v], rows_v, sem).wait()
                pltpu.sync_copy(rows_v, out.at[pl.ds(off, _CH)])
                return 0

            lax.fori_loop(0, _MPCH, chunk, 0)

        one(hdf, idxd, gd)
        one(hsf, idxs, gs)

    return mp_gather


def _dot(a, b):
    return jnp.dot(a, b, preferred_element_type=jnp.float32)


def _lin_body(xd, xs, wd, bd, ws, bs, od, os_):
    od[...] = jnp.maximum(_dot(xd[...], wd[...]) + bd[...], 0.0)
    os_[...] = jnp.maximum(_dot(xs[...], ws[...]) + bs[...], 0.0)


def _row_spec(r, ncols=_D):
    return pl.BlockSpec((r, ncols), lambda i: (i, 0))


def _full_spec(shape):
    return pl.BlockSpec(shape, lambda i: tuple(0 for _ in shape))


def _lin(xd, xs, wd, bd, ws, bs):
    return pl.pallas_call(
        _lin_body,
        grid=(16,),
        in_specs=[_row_spec(_RPT), _row_spec(_RPT),
                  _full_spec((_D, _D)), _full_spec((1, _D)),
                  _full_spec((_D, _D)), _full_spec((1, _D))],
        out_specs=[_row_spec(_RPT), _row_spec(_RPT)],
        out_shape=[jax.ShapeDtypeStruct((_NP, _D), jnp.float32)] * 2,
    )(xd, xs, wd, bd, ws, bs)


def _mm_body(hd, hs, wdd, wrd, wrr, wsd, wss, *outs):
    d = hd[...]
    s = hs[...]
    vals = [_dot(s, wdd[...]), _dot(d, wrd[...]), _dot(d, wrr[...]),
            _dot(d, wsd[...]), _dot(s, wss[...])]
    for j, v in enumerate(vals):
        outs[2 * j][...] = v[:, :_HD]
        outs[2 * j + 1][...] = v[:, _HD:]


def _layer_mm(hd, hs, wdd, wrd, wrr, wsd, wss):
    return pl.pallas_call(
        _mm_body,
        grid=(16,),
        in_specs=[_row_spec(_RPT), _row_spec(_RPT)] + [_full_spec((_D, _D))] * 5,
        out_specs=[_row_spec(_RPT, _HD)] * 10,
        out_shape=[jax.ShapeDtypeStruct((_NP, _HD), jnp.float32)] * 10,
    )(hd, hs, wdd, wrd, wrr, wsd, wss)


def _relu_body(al, ar, bl, br, oa, ob):
    oa[...] = jnp.concatenate(
        [jnp.maximum(al[...], 0.0), jnp.maximum(ar[...], 0.0)], axis=1)
    ob[...] = jnp.concatenate(
        [jnp.maximum(bl[...], 0.0), jnp.maximum(br[...], 0.0)], axis=1)


def _relu_cat(al, ar, bl, br):
    return pl.pallas_call(
        _relu_body,
        grid=(16,),
        in_specs=[_row_spec(_RPT, _HD)] * 4,
        out_specs=[_row_spec(_RPT)] * 2,
        out_shape=[jax.ShapeDtypeStruct((_NP, _D), jnp.float32)] * 2,
    )(al, ar, bl, br)


def _attn_pool(h0, h1, h2, wa, va):
    hs = [h0[...], h1[...], h2[...]]
    va_col = va[...].reshape(_D, 1)
    es = [_dot(jnp.tanh(_dot(h, wa[...])), va_col) for h in hs]
    m = jnp.maximum(jnp.maximum(es[0], es[1]), es[2])
    ws = [jnp.exp(e - m) for e in es]
    tot = ws[0] + ws[1] + ws[2]
    return (ws[0] * hs[0] + ws[1] * hs[1] + ws[2] * hs[2]) / tot


def _attn_body(d0, d1, d2, s0, s1, s2, wad, vad, was, vas, od, os_):
    od[...] = _attn_pool(d0, d1, d2, wad, vad)
    os_[...] = _attn_pool(s0, s1, s2, was, vas)


def _layer_attn(d0, d1, d2, s0, s1, s2, wad, vad, was, vas):
    return pl.pallas_call(
        _attn_body,
        grid=(16,),
        in_specs=[_row_spec(_RPT)] * 6
        + [_full_spec((_D, _D)), _full_spec((1, _D)),
           _full_spec((_D, _D)), _full_spec((1, _D))],
        out_specs=[_row_spec(_RPT)] * 2,
        out_shape=[jax.ShapeDtypeStruct((_NP, _D), jnp.float32)] * 2,
    )(d0, d1, d2, s0, s1, s2, wad, vad, was, vas)


_MB = 256


def _mil_body(gd, gs, wagg, vmil, wmil, attn_o, bag_o):
    g = (gd[...] + gs[...]).reshape(_MB * _BAG, _D)
    ins = jnp.maximum(_dot(g, wagg[...]), 0.0)
    t3 = jnp.tanh(_dot(ins, vmil[...])).reshape(_MB, _BAG, _D)
    ins3 = ins.reshape(_MB, _BAG, _D)
    w_col = wmil[...].reshape(_D, 1)
    cols = [_dot(t3[:, k, :], w_col) for k in range(_BAG)]
    al = jnp.concatenate(cols, axis=1)
    m = jnp.max(al, axis=1, keepdims=True)
    e = jnp.exp(al - m)
    attn = e / jnp.sum(e, axis=1, keepdims=True)
    attn_o[...] = attn
    bag = attn[:, 0:1] * ins3[:, 0, :]
    for k in range(1, _BAG):
        bag = bag + attn[:, k:k + 1] * ins3[:, k, :]
    bag_o[...] = bag


def _mil(gd3, gs3, wagg, vmil, wmil):
    return pl.pallas_call(
        _mil_body,
        grid=(_NB // _MB,),
        in_specs=[pl.BlockSpec((_MB, _BAG, _D), lambda i: (i, 0, 0))] * 2
        + [_full_spec((_D, _D)), _full_spec((_D, _D)), _full_spec((1, _D))],
        out_specs=[pl.BlockSpec((_MB, _BAG), lambda i: (i, 0)),
                   pl.BlockSpec((_MB, _D), lambda i: (i, 0))],
        out_shape=[jax.ShapeDtypeStruct((_NB, _BAG), jnp.float32),
                   jax.ShapeDtypeStruct((_NB, _D), jnp.float32)],
    )(gd3, gs3, wagg, vmil, wmil)


def _llm_body(x, w, b, o):
    k = pl.program_id(0)

    @pl.when(k == 0)
    def _():
        o[...] = jnp.broadcast_to(b[...], (_NB, _D))

    o[...] += _dot(x[...], w[...])

    @pl.when(k == _NKB - 1)
    def _():
        y = o[...]
        n = jnp.sqrt(jnp.sum(y * y, axis=1, keepdims=True)) + 1e-12
        o[...] = y / n


def _llm_head(x, w, b):
    return pl.pallas_call(
        _llm_body,
        grid=(_NKB,),
        in_specs=[pl.BlockSpec((_NB, _KB), lambda k: (0, k)),
                  pl.BlockSpec((_KB, _D), lambda k: (k, 0)),
                  _full_spec((1, _D))],
        out_specs=pl.BlockSpec((_NB, _D), lambda k: (0, 0)),
        out_shape=jax.ShapeDtypeStruct((_NB, _D), jnp.float32),
    )(x, w, b)


def _head_body(bag, llm_n, w1, b1, w2, b2, o):
    kg = bag[...]
    n = jnp.sqrt(jnp.sum(kg * kg, axis=1, keepdims=True)) + 1e-12
    kgn = kg / n
    w1v = w1[...]
    h = _dot(kgn, w1v[:_D, :]) + _dot(llm_n[...], w1v[_D:, :]) + b1[...]
    h = jnp.maximum(h, 0.0)
    pred = _dot(h, w2[...]) + b2[0, 0]
    o[...] = jnp.broadcast_to(pred, (_NB, _D))


def _head(bag, llm_n, w1, b1, w2, b2):
    return pl.pallas_call(
        _head_body,
        grid=(1,),
        in_specs=[_full_spec((_NB, _D)), _full_spec((_NB, _D)),
                  _full_spec((2 * _D, _D)), _full_spec((1, _D)),
                  _full_spec((_D, 1)), _full_spec((1, 1))],
        out_specs=_full_spec((_NB, _D)),
        out_shape=jax.ShapeDtypeStruct((_NB, _D), jnp.float32),
    )(bag, llm_n, w1, b1, w2, b2)


def kernel(drug_feat, disease_feat, edge_dd, edge_rd, edge_rr, mp_ins,
           llm_rep, W_lin_drug, b_lin_drug, W_lin_dis, b_lin_dis, W_dd,
           W_rd, W_rr, W_self_drug, W_self_dis, Wa_drug, va_drug, Wa_dis,
           va_dis, W_agg, V_mil, w_mil, W_llm, b_llm, W_mlp1, b_mlp1,
           W_mlp2, b_mlp2):
    rowpad = ((0, _NP - _N), (0, 0))
    dfp = jnp.pad(drug_feat, rowpad)
    sfp = jnp.pad(disease_feat, rowpad)

    def pad_edges(e):
        src = jnp.pad(e[0], (0, _EP - _E)).reshape(16, _NCH, _CH)
        dst = jnp.pad(e[1], (0, _EP - _E),
                      constant_values=_N + 80).reshape(16, _NCH, _CH)
        return src, dst

    dd_s, dd_d = pad_edges(edge_dd)
    rd_s, rd_d = pad_edges(edge_rd)
    rr_s, rr_d = pad_edges(edge_rr)

    row = lambda v: v.reshape(1, -1)

    hd, hs = _lin(dfp, sfp, W_lin_drug, row(b_lin_drug),
                  W_lin_dis, row(b_lin_dis))
    drugs = [hd]
    diss = [hs]
    for l in range(2):
        (tddL, tddR, trdL, trdR, trrL, trrR,
         sdL, sdR, ssL, ssR) = _layer_mm(
            hd, hs, W_dd[l], W_rd[l], W_rr[l], W_self_drug[l], W_self_dis[l])
        msL, msR = _make_segsum(2)(ssL, ssR, tddL, tddR, dd_s, dd_d,
                                   trdL, trdR, rd_s, rd_d)
        mdL, mdR = _make_segsum(1)(sdL, sdR, trrL, trrR, rr_s, rr_d)
        hd, hs = _relu_cat(mdL, mdR, msL, msR)
        drugs.append(hd)
        diss.append(hs)

    hdf, hsf = _layer_attn(drugs[0], drugs[1], drugs[2],
                           diss[0], diss[1], diss[2],
                           Wa_drug, row(va_drug), Wa_dis, row(va_dis))

    idxd = mp_ins[..., 0].reshape(-1)
    idxs = mp_ins[..., 1].reshape(-1)
    gd, gs = _make_mp_gather()(hdf, hsf, idxd, idxs)

    attn, bag = _mil(gd.reshape(_NB, _BAG, _D), gs.reshape(_NB, _BAG, _D),
                     W_agg, V_mil, row(w_mil))

    llm_n = _llm_head(llm_rep, W_llm, row(b_llm))
    pred_full = _head(bag, llm_n, W_mlp1, row(b_mlp1), W_mlp2,
                      b_mlp2.reshape(1, 1))
    return pred_full[:, :1], attn


# speedup vs baseline: 3.5597x; 1.1768x over previous
import functools

import jax
import jax.numpy as jnp
from jax import lax
from jax.experimental import pallas as pl
from jax.experimental.pallas import tpu as pltpu
from jax.experimental.pallas import tpu_sc as plsc

_N = 25000
_NP = 25088
_RPT = _NP // 16
_D = 128
_HD = 64
_E = 400000
_CH = 256
_EP = 401408
_EPT = _EP // 16
_NCH = _EPT // _CH
_NB = 1024
_BAG = 16
_LLM_D = 32000
_KB = 3200
_NKB = _LLM_D // _KB

def _get_mesh():
    return plsc.VectorSubcoreMesh(core_axis_name="c", subcore_axis_name="s",
                                  num_cores=2, num_subcores=16)


@functools.lru_cache(maxsize=None)
def _make_segsum(n_phases):
    n_in = 2 + 4 * n_phases
    nbuf = 1
    sec = 7
    nsec = _NCH // sec

    @functools.partial(
        pl.kernel,
        out_type=[jax.ShapeDtypeStruct((_NP, _HD), jnp.float32),
                  jax.ShapeDtypeStruct((_NP, _HD), jnp.float32)],
        mesh=_get_mesh(),
        compiler_params=pltpu.CompilerParams(use_tc_tiling_on_sc=False),
        scratch_types=[
            pltpu.VMEM((sec, _CH), jnp.int32),
            pltpu.VMEM((sec, _CH), jnp.int32),
        ]
        + [pltpu.VMEM((_CH, _HD), jnp.float32) for _ in range(nbuf)]
        + [pltpu.SemaphoreType.DMA for _ in range(nbuf)]
        + [pltpu.VMEM_SHARED((_NP, _HD), jnp.float32)],
    )
    def segsum(*refs):
        ins = refs[:n_in]
        out_l, out_r = refs[n_in], refs[n_in + 1]
        src_v, dst_v = refs[n_in + 2], refs[n_in + 3]
        bufs = refs[n_in + 4:n_in + 4 + nbuf]
        sems = refs[n_in + 4 + nbuf:n_in + 4 + 2 * nbuf]
        acc = refs[n_in + 4 + 2 * nbuf]
        init_l, init_r = ins[0], ins[1]
        c = lax.axis_index("c")
        t = lax.axis_index("s")
        r0 = t * _RPT

        def run(init, tabs, out):
            pltpu.sync_copy(init.at[pl.ds(r0, _RPT)], acc.at[pl.ds(r0, _RPT)])
            plsc.subcore_barrier()
            for ph in range(n_phases):
                tab = tabs[ph]
                src = ins[2 + 4 * ph + 2]
                dst = ins[2 + 4 * ph + 3]

                def section(s, _):
                    pltpu.sync_copy(src.at[t, pl.ds(s * sec, sec)], src_v)
                    pltpu.sync_copy(dst.at[t, pl.ds(s * sec, sec)], dst_v)

                    def chunk(k, _):
                        pltpu.async_copy(tab.at[src_v.at[k]], bufs[0],
                                         sems[0]).wait()
                        pltpu.sync_copy(bufs[0], acc.at[dst_v.at[k]],
                                        add=True)
                        return 0

                    lax.fori_loop(0, sec, chunk, 0)
                    return 0

                lax.fori_loop(0, nsec, section, 0)
            plsc.subcore_barrier()
            pltpu.sync_copy(acc.at[pl.ds(r0, _RPT)], out.at[pl.ds(r0, _RPT)])

        @pl.when(c == 0)
        def _():
            run(init_l, [ins[2 + 4 * p] for p in range(n_phases)], out_l)

        @pl.when(c == 1)
        def _():
            run(init_r, [ins[2 + 4 * p + 1] for p in range(n_phases)], out_r)

    return segsum


_MPW = (_NB * _BAG) // 32
_MPCH = _MPW // _CH


@functools.lru_cache(maxsize=None)
def _make_mp_gather():
    @functools.partial(
        pl.kernel,
        out_type=[jax.ShapeDtypeStruct((_NB * _BAG, _D), jnp.float32),
                  jax.ShapeDtypeStruct((_NB * _BAG, _D), jnp.float32)],
        mesh=_get_mesh(),
        scratch_types=[
            pltpu.VMEM((_CH,), jnp.int32),
            pltpu.VMEM((_CH, _D), jnp.float32),
            pltpu.SemaphoreType.DMA,
        ],
    )
    def mp_gather(hdf, hsf, idxd, idxs, gd, gs, idx_v, rows_v, sem):
        c = lax.axis_index("c")
        s = lax.axis_index("s")
        wid = s * 2 + c

        def one(tab, idx, out):
            def chunk(i, _):
                off = wid * _MPW + i * _CH
                pltpu.sync_copy(idx.at[pl.ds(off, _CH)], idx_v)
                pltpu.async_copy(tab.at[idx_v], rows_v, sem).wait()
                pltpu.sync_copy(rows_v, out.at[pl.ds(off, _CH)])
                return 0

            lax.fori_loop(0, _MPCH, chunk, 0)

        one(hdf, idxd, gd)
        one(hsf, idxs, gs)

    return mp_gather


def _dot(a, b):
    return jnp.dot(a, b, preferred_element_type=jnp.float32)


def _lin_body(xd, xs, wd, bd, ws, bs, od, os_):
    od[...] = jnp.maximum(_dot(xd[...], wd[...]) + bd[...], 0.0)
    os_[...] = jnp.maximum(_dot(xs[...], ws[...]) + bs[...], 0.0)


def _row_spec(r, ncols=_D):
    return pl.BlockSpec((r, ncols), lambda i: (i, 0))


def _full_spec(shape):
    return pl.BlockSpec(shape, lambda i: tuple(0 for _ in shape))


def _lin(xd, xs, wd, bd, ws, bs):
    return pl.pallas_call(
        _lin_body,
        grid=(16,),
        in_specs=[_row_spec(_RPT), _row_spec(_RPT),
                  _full_spec((_D, _D)), _full_spec((1, _D)),
                  _full_spec((_D, _D)), _full_spec((1, _D))],
        out_specs=[_row_spec(_RPT), _row_spec(_RPT)],
        out_shape=[jax.ShapeDtypeStruct((_NP, _D), jnp.float32)] * 2,
    )(xd, xs, wd, bd, ws, bs)


def _mm_body(hd, hs, wdd, wrd, wrr, wsd, wss, *outs):
    d = hd[...]
    s = hs[...]
    vals = [_dot(s, wdd[...]), _dot(d, wrd[...]), _dot(d, wrr[...]),
            _dot(d, wsd[...]), _dot(s, wss[...])]
    for j, v in enumerate(vals):
        outs[2 * j][...] = v[:, :_HD]
        outs[2 * j + 1][...] = v[:, _HD:]


def _layer_mm(hd, hs, wdd, wrd, wrr, wsd, wss):
    return pl.pallas_call(
        _mm_body,
        grid=(16,),
        in_specs=[_row_spec(_RPT), _row_spec(_RPT)] + [_full_spec((_D, _D))] * 5,
        out_specs=[_row_spec(_RPT, _HD)] * 10,
        out_shape=[jax.ShapeDtypeStruct((_NP, _HD), jnp.float32)] * 10,
    )(hd, hs, wdd, wrd, wrr, wsd, wss)


def _relu_body(al, ar, bl, br, oa, ob):
    oa[...] = jnp.concatenate(
        [jnp.maximum(al[...], 0.0), jnp.maximum(ar[...], 0.0)], axis=1)
    ob[...] = jnp.concatenate(
        [jnp.maximum(bl[...], 0.0), jnp.maximum(br[...], 0.0)], axis=1)


def _relu_cat(al, ar, bl, br):
    return pl.pallas_call(
        _relu_body,
        grid=(16,),
        in_specs=[_row_spec(_RPT, _HD)] * 4,
        out_specs=[_row_spec(_RPT)] * 2,
        out_shape=[jax.ShapeDtypeStruct((_NP, _D), jnp.float32)] * 2,
    )(al, ar, bl, br)


def _attn_pool(h0, h1, h2, wa, va):
    hs = [h0[...], h1[...], h2[...]]
    va_col = va[...].reshape(_D, 1)
    es = [_dot(jnp.tanh(_dot(h, wa[...])), va_col) for h in hs]
    m = jnp.maximum(jnp.maximum(es[0], es[1]), es[2])
    ws = [jnp.exp(e - m) for e in es]
    tot = ws[0] + ws[1] + ws[2]
    return (ws[0] * hs[0] + ws[1] * hs[1] + ws[2] * hs[2]) / tot


def _attn_body(d0, d1, d2, s0, s1, s2, wad, vad, was, vas, od, os_):
    od[...] = _attn_pool(d0, d1, d2, wad, vad)
    os_[...] = _attn_pool(s0, s1, s2, was, vas)


def _layer_attn(d0, d1, d2, s0, s1, s2, wad, vad, was, vas):
    return pl.pallas_call(
        _attn_body,
        grid=(16,),
        in_specs=[_row_spec(_RPT)] * 6
        + [_full_spec((_D, _D)), _full_spec((1, _D)),
           _full_spec((_D, _D)), _full_spec((1, _D))],
        out_specs=[_row_spec(_RPT)] * 2,
        out_shape=[jax.ShapeDtypeStruct((_NP, _D), jnp.float32)] * 2,
    )(d0, d1, d2, s0, s1, s2, wad, vad, was, vas)


_MB = 256


def _mil_body(gd, gs, wagg, vmil, wmil, attn_o, bag_o):
    g = (gd[...] + gs[...]).reshape(_MB * _BAG, _D)
    ins = jnp.maximum(_dot(g, wagg[...]), 0.0)
    t3 = jnp.tanh(_dot(ins, vmil[...])).reshape(_MB, _BAG, _D)
    ins3 = ins.reshape(_MB, _BAG, _D)
    w_col = wmil[...].reshape(_D, 1)
    cols = [_dot(t3[:, k, :], w_col) for k in range(_BAG)]
    al = jnp.concatenate(cols, axis=1)
    m = jnp.max(al, axis=1, keepdims=True)
    e = jnp.exp(al - m)
    attn = e / jnp.sum(e, axis=1, keepdims=True)
    attn_o[...] = attn
    bag = attn[:, 0:1] * ins3[:, 0, :]
    for k in range(1, _BAG):
        bag = bag + attn[:, k:k + 1] * ins3[:, k, :]
    bag_o[...] = bag


def _mil(gd3, gs3, wagg, vmil, wmil):
    return pl.pallas_call(
        _mil_body,
        grid=(_NB // _MB,),
        in_specs=[pl.BlockSpec((_MB, _BAG, _D), lambda i: (i, 0, 0))] * 2
        + [_full_spec((_D, _D)), _full_spec((_D, _D)), _full_spec((1, _D))],
        out_specs=[pl.BlockSpec((_MB, _BAG), lambda i: (i, 0)),
                   pl.BlockSpec((_MB, _D), lambda i: (i, 0))],
        out_shape=[jax.ShapeDtypeStruct((_NB, _BAG), jnp.float32),
                   jax.ShapeDtypeStruct((_NB, _D), jnp.float32)],
    )(gd3, gs3, wagg, vmil, wmil)


def _llm_body(x, w, b, o):
    k = pl.program_id(0)

    @pl.when(k == 0)
    def _():
        o[...] = jnp.broadcast_to(b[...], (_NB, _D))

    o[...] += _dot(x[...], w[...])

    @pl.when(k == _NKB - 1)
    def _():
        y = o[...]
        n = jnp.sqrt(jnp.sum(y * y, axis=1, keepdims=True)) + 1e-12
        o[...] = y / n


def _llm_head(x, w, b):
    return pl.pallas_call(
        _llm_body,
        grid=(_NKB,),
        in_specs=[pl.BlockSpec((_NB, _KB), lambda k: (0, k)),
                  pl.BlockSpec((_KB, _D), lambda k: (k, 0)),
                  _full_spec((1, _D))],
        out_specs=pl.BlockSpec((_NB, _D), lambda k: (0, 0)),
        out_shape=jax.ShapeDtypeStruct((_NB, _D), jnp.float32),
    )(x, w, b)


def _head_body(bag, llm_n, w1, b1, w2, b2, o):
    kg = bag[...]
    n = jnp.sqrt(jnp.sum(kg * kg, axis=1, keepdims=True)) + 1e-12
    kgn = kg / n
    w1v = w1[...]
    h = _dot(kgn, w1v[:_D, :]) + _dot(llm_n[...], w1v[_D:, :]) + b1[...]
    h = jnp.maximum(h, 0.0)
    pred = _dot(h, w2[...]) + b2[0, 0]
    o[...] = jnp.broadcast_to(pred, (_NB, _D))


def _head(bag, llm_n, w1, b1, w2, b2):
    return pl.pallas_call(
        _head_body,
        grid=(1,),
        in_specs=[_full_spec((_NB, _D)), _full_spec((_NB, _D)),
                  _full_spec((2 * _D, _D)), _full_spec((1, _D)),
                  _full_spec((_D, 1)), _full_spec((1, 1))],
        out_specs=_full_spec((_NB, _D)),
        out_shape=jax.ShapeDtypeStruct((_NB, _D), jnp.float32),
    )(bag, llm_n, w1, b1, w2, b2)


def kernel(drug_feat, disease_feat, edge_dd, edge_rd, edge_rr, mp_ins,
           llm_rep, W_lin_drug, b_lin_drug, W_lin_dis, b_lin_dis, W_dd,
           W_rd, W_rr, W_self_drug, W_self_dis, Wa_drug, va_drug, Wa_dis,
           va_dis, W_agg, V_mil, w_mil, W_llm, b_llm, W_mlp1, b_mlp1,
           W_mlp2, b_mlp2):
    rowpad = ((0, _NP - _N), (0, 0))
    dfp = jnp.pad(drug_feat, rowpad)
    sfp = jnp.pad(disease_feat, rowpad)

    def pad_edges(e):
        src = jnp.pad(e[0], (0, _EP - _E)).reshape(16, _NCH, _CH)
        dst = jnp.pad(e[1], (0, _EP - _E),
                      constant_values=_N + 80).reshape(16, _NCH, _CH)
        return src, dst

    dd_s, dd_d = pad_edges(edge_dd)
    rd_s, rd_d = pad_edges(edge_rd)
    rr_s, rr_d = pad_edges(edge_rr)

    row = lambda v: v.reshape(1, -1)

    hd, hs = _lin(dfp, sfp, W_lin_drug, row(b_lin_drug),
                  W_lin_dis, row(b_lin_dis))
    drugs = [hd]
    diss = [hs]
    for l in range(2):
        (tddL, tddR, trdL, trdR, trrL, trrR,
         sdL, sdR, ssL, ssR) = _layer_mm(
            hd, hs, W_dd[l], W_rd[l], W_rr[l], W_self_drug[l], W_self_dis[l])
        msL, msR = _make_segsum(2)(ssL, ssR, tddL, tddR, dd_s, dd_d,
                                   trdL, trdR, rd_s, rd_d)
        mdL, mdR = _make_segsum(1)(sdL, sdR, trrL, trrR, rr_s, rr_d)
        hd, hs = _relu_cat(mdL, mdR, msL, msR)
        drugs.append(hd)
        diss.append(hs)

    hdf, hsf = _layer_attn(drugs[0], drugs[1], drugs[2],
                           diss[0], diss[1], diss[2],
                           Wa_drug, row(va_drug), Wa_dis, row(va_dis))

    idxd = mp_ins[..., 0].reshape(-1)
    idxs = mp_ins[..., 1].reshape(-1)
    gd, gs = _make_mp_gather()(hdf, hsf, idxd, idxs)

    attn, bag = _mil(gd.reshape(_NB, _BAG, _D), gs.reshape(_NB, _BAG, _D),
                     W_agg, V_mil, row(w_mil))

    llm_n = _llm_head(llm_rep, W_llm, row(b_llm))
    pred_full = _head(bag, llm_n, W_mlp1, row(b_mlp1), W_mlp2,
                      b_mlp2.reshape(1, 1))
    return pred_full[:, :1], attn


# speedup vs baseline: 3.6615x; 1.0286x over previous
import functools

import jax
import jax.numpy as jnp
from jax import lax
from jax.experimental import pallas as pl
from jax.experimental.pallas import tpu as pltpu
from jax.experimental.pallas import tpu_sc as plsc

_N = 25000
_NP = 25088
_RPT = _NP // 16
_D = 128
_HD = 64
_E = 400000
_CH = 256
_EP = 401408
_EPT = _EP // 16
_NCH = _EPT // _CH
_NB = 1024
_BAG = 16
_LLM_D = 32000
_KB = 3200
_NKB = _LLM_D // _KB

def _get_mesh():
    return plsc.VectorSubcoreMesh(core_axis_name="c", subcore_axis_name="s",
                                  num_cores=2, num_subcores=16)


@functools.lru_cache(maxsize=None)
def _make_segsum(n_phases):
    n_in = 2 + 4 * n_phases
    nbuf = 1
    sec = 7
    nsec = _NCH // sec

    @functools.partial(
        pl.kernel,
        out_type=[jax.ShapeDtypeStruct((_NP, _HD), jnp.float32),
                  jax.ShapeDtypeStruct((_NP, _HD), jnp.float32)],
        mesh=_get_mesh(),
        compiler_params=pltpu.CompilerParams(use_tc_tiling_on_sc=False),
        scratch_types=[
            pltpu.VMEM((sec, _CH), jnp.int32),
            pltpu.VMEM((sec, _CH), jnp.int32),
        ]
        + [pltpu.VMEM((_CH, _HD), jnp.float32) for _ in range(nbuf)]
        + [pltpu.SemaphoreType.DMA for _ in range(nbuf)]
        + [pltpu.VMEM_SHARED((_NP, _HD), jnp.float32)],
    )
    def segsum(*refs):
        ins = refs[:n_in]
        out_l, out_r = refs[n_in], refs[n_in + 1]
        src_v, dst_v = refs[n_in + 2], refs[n_in + 3]
        bufs = refs[n_in + 4:n_in + 4 + nbuf]
        sems = refs[n_in + 4 + nbuf:n_in + 4 + 2 * nbuf]
        acc = refs[n_in + 4 + 2 * nbuf]
        init_l, init_r = ins[0], ins[1]
        c = lax.axis_index("c")
        t = lax.axis_index("s")
        r0 = t * _RPT

        def run(init, tabs, out):
            pltpu.sync_copy(init.at[pl.ds(r0, _RPT)], acc.at[pl.ds(r0, _RPT)])
            plsc.subcore_barrier()
            for ph in range(n_phases):
                tab = tabs[ph]
                src = ins[2 + 4 * ph + 2]
                dst = ins[2 + 4 * ph + 3]

                def section(s, _):
                    pltpu.sync_copy(src.at[t, pl.ds(s * sec, sec)], src_v)
                    pltpu.sync_copy(dst.at[t, pl.ds(s * sec, sec)], dst_v)

                    def chunk(k, _):
                        pltpu.async_copy(tab.at[src_v.at[k]], bufs[0],
                                         sems[0]).wait()
                        pltpu.sync_copy(bufs[0], acc.at[dst_v.at[k]],
                                        add=True)
                        return 0

                    lax.fori_loop(0, sec, chunk, 0)
                    return 0

                lax.fori_loop(0, nsec, section, 0)
            plsc.subcore_barrier()
            pltpu.sync_copy(acc.at[pl.ds(r0, _RPT)], out.at[pl.ds(r0, _RPT)])

        @pl.when(c == 0)
        def _():
            run(init_l, [ins[2 + 4 * p] for p in range(n_phases)], out_l)

        @pl.when(c == 1)
        def _():
            run(init_r, [ins[2 + 4 * p + 1] for p in range(n_phases)], out_r)

    return segsum


_MPW = (_NB * _BAG) // 32
_MPCH = _MPW // _CH


@functools.lru_cache(maxsize=None)
def _make_mp_gather():
    @functools.partial(
        pl.kernel,
        out_type=[jax.ShapeDtypeStruct((_NB * _BAG, _D), jnp.float32),
                  jax.ShapeDtypeStruct((_NB * _BAG, _D), jnp.float32)],
        mesh=_get_mesh(),
        scratch_types=[
            pltpu.VMEM((_CH,), jnp.int32),
            pltpu.VMEM((_CH, _D), jnp.float32),
            pltpu.SemaphoreType.DMA,
        ],
    )
    def mp_gather(hdf, hsf, idxd, idxs, gd, gs, idx_v, rows_v, sem):
        c = lax.axis_index("c")
        s = lax.axis_index("s")
        wid = s * 2 + c

        def one(tab, idx, out):
            def chunk(i, _):
                off = wid * _MPW + i * _CH
                pltpu.sync_copy(idx.at[pl.ds(off, _CH)], idx_v)
                pltpu.async_copy(tab.at[idx_v], rows_v, sem).wait()
                pltpu.sync_copy(rows_v, out.at[pl.ds(off, _CH)])
                return 0

            lax.fori_loop(0, _MPCH, chunk, 0)

        one(hdf, idxd, gd)
        one(hsf, idxs, gs)

    return mp_gather


def _dot(a, b):
    return jnp.dot(a, b, preferred_element_type=jnp.float32)


def _lin_body(xd, xs, wd, bd, ws, bs, od, os_):
    od[...] = jnp.maximum(_dot(xd[...], wd[...]) + bd[...], 0.0)
    os_[...] = jnp.maximum(_dot(xs[...], ws[...]) + bs[...], 0.0)


def _row_spec(r, ncols=_D):
    return pl.BlockSpec((r, ncols), lambda i: (i, 0))


def _full_spec(shape):
    return pl.BlockSpec(shape, lambda i: tuple(0 for _ in shape))


def _lin(xd, xs, wd, bd, ws, bs):
    return pl.pallas_call(
        _lin_body,
        grid=(16,),
        in_specs=[_row_spec(_RPT), _row_spec(_RPT),
                  _full_spec((_D, _D)), _full_spec((1, _D)),
                  _full_spec((_D, _D)), _full_spec((1, _D))],
        out_specs=[_row_spec(_RPT), _row_spec(_RPT)],
        out_shape=[jax.ShapeDtypeStruct((_NP, _D), jnp.float32)] * 2,
    )(xd, xs, wd, bd, ws, bs)


def _halves_in(hd, hs):
    def load(x):
        if isinstance(x, tuple):
            return jnp.concatenate(
                [jnp.maximum(x[0][...], 0.0), jnp.maximum(x[1][...], 0.0)],
                axis=1)
        return x[...]
    return load(hd), load(hs)


def _mm_body_full(hd, hs, wdd, wrd, wrr, wsd, wss, *outs):
    _mm_common(hd[...], hs[...], wdd, wrd, wrr, wsd, wss, outs)


def _mm_body_halves(hdl, hdr, hsl, hsr, wdd, wrd, wrr, wsd, wss, *outs):
    d, s = _halves_in((hdl, hdr), (hsl, hsr))
    _mm_common(d, s, wdd, wrd, wrr, wsd, wss, outs)


def _mm_common(d, s, wdd, wrd, wrr, wsd, wss, outs):
    vals = [_dot(s, wdd[...]), _dot(d, wrd[...]), _dot(d, wrr[...]),
            _dot(d, wsd[...]), _dot(s, wss[...])]
    for j, v in enumerate(vals):
        outs[2 * j][...] = v[:, :_HD]
        outs[2 * j + 1][...] = v[:, _HD:]


def _layer_mm(hd, hs, wdd, wrd, wrr, wsd, wss):
    halved = isinstance(hd, tuple)
    body = _mm_body_halves if halved else _mm_body_full
    h_in = ([_row_spec(_RPT, _HD)] * 4 if halved
            else [_row_spec(_RPT), _row_spec(_RPT)])
    h_args = (hd + hs) if halved else (hd, hs)
    return pl.pallas_call(
        body,
        grid=(16,),
        in_specs=h_in + [_full_spec((_D, _D))] * 5,
        out_specs=[_row_spec(_RPT, _HD)] * 10,
        out_shape=[jax.ShapeDtypeStruct((_NP, _HD), jnp.float32)] * 10,
    )(*h_args, wdd, wrd, wrr, wsd, wss)


def _attn_pool(hs, wa, va):
    va_col = va[...].reshape(_D, 1)
    es = [_dot(jnp.tanh(_dot(h, wa[...])), va_col) for h in hs]
    m = jnp.maximum(jnp.maximum(es[0], es[1]), es[2])
    ws = [jnp.exp(e - m) for e in es]
    tot = ws[0] + ws[1] + ws[2]
    return (ws[0] * hs[0] + ws[1] * hs[1] + ws[2] * hs[2]) / tot


def _attn_body(d0, d1l, d1r, d2l, d2r, s0, s1l, s1r, s2l, s2r,
               wad, vad, was, vas, od, os_):
    d1, s1 = _halves_in((d1l, d1r), (s1l, s1r))
    d2, s2 = _halves_in((d2l, d2r), (s2l, s2r))
    od[...] = _attn_pool([d0[...], d1, d2], wad, vad)
    os_[...] = _attn_pool([s0[...], s1, s2], was, vas)


def _layer_attn(d0, d1, d2, s0, s1, s2, wad, vad, was, vas):
    return pl.pallas_call(
        _attn_body,
        grid=(16,),
        in_specs=[_row_spec(_RPT)] + [_row_spec(_RPT, _HD)] * 4
        + [_row_spec(_RPT)] + [_row_spec(_RPT, _HD)] * 4
        + [_full_spec((_D, _D)), _full_spec((1, _D)),
           _full_spec((_D, _D)), _full_spec((1, _D))],
        out_specs=[_row_spec(_RPT)] * 2,
        out_shape=[jax.ShapeDtypeStruct((_NP, _D), jnp.float32)] * 2,
    )(d0, *d1, *d2, s0, *s1, *s2, wad, vad, was, vas)


_MB = 256


def _mil_body(gd, gs, wagg, vmil, wmil, attn_o, bag_o):
    g = (gd[...] + gs[...]).reshape(_MB * _BAG, _D)
    ins = jnp.maximum(_dot(g, wagg[...]), 0.0)
    t3 = jnp.tanh(_dot(ins, vmil[...])).reshape(_MB, _BAG, _D)
    ins3 = ins.reshape(_MB, _BAG, _D)
    w_col = wmil[...].reshape(_D, 1)
    cols = [_dot(t3[:, k, :], w_col) for k in range(_BAG)]
    al = jnp.concatenate(cols, axis=1)
    m = jnp.max(al, axis=1, keepdims=True)
    e = jnp.exp(al - m)
    attn = e / jnp.sum(e, axis=1, keepdims=True)
    attn_o[...] = attn
    bag = attn[:, 0:1] * ins3[:, 0, :]
    for k in range(1, _BAG):
        bag = bag + attn[:, k:k + 1] * ins3[:, k, :]
    bag_o[...] = bag


def _mil(gd3, gs3, wagg, vmil, wmil):
    return pl.pallas_call(
        _mil_body,
        grid=(_NB // _MB,),
        in_specs=[pl.BlockSpec((_MB, _BAG, _D), lambda i: (i, 0, 0))] * 2
        + [_full_spec((_D, _D)), _full_spec((_D, _D)), _full_spec((1, _D))],
        out_specs=[pl.BlockSpec((_MB, _BAG), lambda i: (i, 0)),
                   pl.BlockSpec((_MB, _D), lambda i: (i, 0))],
        out_shape=[jax.ShapeDtypeStruct((_NB, _BAG), jnp.float32),
                   jax.ShapeDtypeStruct((_NB, _D), jnp.float32)],
    )(gd3, gs3, wagg, vmil, wmil)


def _llm_body(x, w, b, o):
    k = pl.program_id(0)

    @pl.when(k == 0)
    def _():
        o[...] = jnp.broadcast_to(b[...], (_NB, _D))

    o[...] += _dot(x[...], w[...])

    @pl.when(k == _NKB - 1)
    def _():
        y = o[...]
        n = jnp.sqrt(jnp.sum(y * y, axis=1, keepdims=True)) + 1e-12
        o[...] = y / n


def _llm_head(x, w, b):
    return pl.pallas_call(
        _llm_body,
        grid=(_NKB,),
        in_specs=[pl.BlockSpec((_NB, _KB), lambda k: (0, k)),
                  pl.BlockSpec((_KB, _D), lambda k: (k, 0)),
                  _full_spec((1, _D))],
        out_specs=pl.BlockSpec((_NB, _D), lambda k: (0, 0)),
        out_shape=jax.ShapeDtypeStruct((_NB, _D), jnp.float32),
    )(x, w, b)


def _head_body(bag, llm_n, w1, b1, w2, b2, o):
    kg = bag[...]
    n = jnp.sqrt(jnp.sum(kg * kg, axis=1, keepdims=True)) + 1e-12
    kgn = kg / n
    w1v = w1[...]
    h = _dot(kgn, w1v[:_D, :]) + _dot(llm_n[...], w1v[_D:, :]) + b1[...]
    h = jnp.maximum(h, 0.0)
    pred = _dot(h, w2[...]) + b2[0, 0]
    o[...] = jnp.broadcast_to(pred, (_NB, _D))


def _head(bag, llm_n, w1, b1, w2, b2):
    return pl.pallas_call(
        _head_body,
        grid=(1,),
        in_specs=[_full_spec((_NB, _D)), _full_spec((_NB, _D)),
                  _full_spec((2 * _D, _D)), _full_spec((1, _D)),
                  _full_spec((_D, 1)), _full_spec((1, 1))],
        out_specs=_full_spec((_NB, _D)),
        out_shape=jax.ShapeDtypeStruct((_NB, _D), jnp.float32),
    )(bag, llm_n, w1, b1, w2, b2)


def kernel(drug_feat, disease_feat, edge_dd, edge_rd, edge_rr, mp_ins,
           llm_rep, W_lin_drug, b_lin_drug, W_lin_dis, b_lin_dis, W_dd,
           W_rd, W_rr, W_self_drug, W_self_dis, Wa_drug, va_drug, Wa_dis,
           va_dis, W_agg, V_mil, w_mil, W_llm, b_llm, W_mlp1, b_mlp1,
           W_mlp2, b_mlp2):
    rowpad = ((0, _NP - _N), (0, 0))
    dfp = jnp.pad(drug_feat, rowpad)
    sfp = jnp.pad(disease_feat, rowpad)

    def pad_edges(e):
        src = jnp.pad(e[0], (0, _EP - _E)).reshape(16, _NCH, _CH)
        dst = jnp.pad(e[1], (0, _EP - _E),
                      constant_values=_N + 80).reshape(16, _NCH, _CH)
        return src, dst

    dd_s, dd_d = pad_edges(edge_dd)
    rd_s, rd_d = pad_edges(edge_rd)
    rr_s, rr_d = pad_edges(edge_rr)

    row = lambda v: v.reshape(1, -1)

    hd, hs = _lin(dfp, sfp, W_lin_drug, row(b_lin_drug),
                  W_lin_dis, row(b_lin_dis))
    drugs = [hd]
    diss = [hs]
    for l in range(2):
        (tddL, tddR, trdL, trdR, trrL, trrR,
         sdL, sdR, ssL, ssR) = _layer_mm(
            hd, hs, W_dd[l], W_rd[l], W_rr[l], W_self_drug[l], W_self_dis[l])
        msL, msR = _make_segsum(2)(ssL, ssR, tddL, tddR, dd_s, dd_d,
                                   trdL, trdR, rd_s, rd_d)
        mdL, mdR = _make_segsum(1)(sdL, sdR, trrL, trrR, rr_s, rr_d)
        hd, hs = (mdL, mdR), (msL, msR)
        drugs.append(hd)
        diss.append(hs)

    hdf, hsf = _layer_attn(drugs[0], drugs[1], drugs[2],
                           diss[0], diss[1], diss[2],
                           Wa_drug, row(va_drug), Wa_dis, row(va_dis))

    idxd = mp_ins[..., 0].reshape(-1)
    idxs = mp_ins[..., 1].reshape(-1)
    gd, gs = _make_mp_gather()(hdf, hsf, idxd, idxs)

    attn, bag = _mil(gd.reshape(_NB, _BAG, _D), gs.reshape(_NB, _BAG, _D),
                     W_agg, V_mil, row(w_mil))

    llm_n = _llm_head(llm_rep, W_llm, row(b_llm))
    pred_full = _head(bag, llm_n, W_mlp1, row(b_mlp1), W_mlp2,
                      b_mlp2.reshape(1, 1))
    return pred_full[:, :1], attn
